# baseline (speedup 1.0000x reference)
import jax
import jax.numpy as jnp
from jax.experimental import pallas as pl
from jax.experimental.pallas import tpu as pltpu

B, T, V, F = 64, 48, 64, 16
QD, NE, PLM = 5, 16, 768
H2 = 2 * F
IN = 2 * F + 1
ALPHA = 1.0

_PREC = jax.lax.Precision.DEFAULT


def _prelude_body(vprT_ref, fW1T_ref, fb1_ref, fW2T_ref, fb2_ref,
                  gW1T_ref, gb1_ref, gW2T_ref, gb2_ref,
                  br_ref, bu_ref, bc_ref, mask_ref,
                  adjod_ref, qvT_ref, bqrT_ref, bquT_ref, bqcT_ref, ivto_ref):
    vprT = vprT_ref[...]
    hfT = jax.nn.relu(jnp.dot(fW1T_ref[...], vprT, precision=_PREC) + fb1_ref[...])
    qvT = jnp.dot(fW2T_ref[...], hfT, precision=_PREC) + fb2_ref[...]
    hgT = jax.nn.relu(jnp.dot(gW1T_ref[...], vprT, precision=_PREC) + gb1_ref[...])
    neT = jnp.dot(gW2T_ref[...], hgT, precision=_PREC) + gb2_ref[...]
    nrm = jnp.maximum(jnp.sqrt(jnp.sum(neT * neT, axis=0, keepdims=True)), 1e-12)
    neT = neT / nrm
    s = jax.lax.dot_general(neT, neT, (((0,), (0,)), ((), ())),
                            precision=_PREC)
    s = s - jnp.max(s, axis=0, keepdims=True)
    e = jnp.exp(s)
    adjT = e / jnp.sum(e, axis=0, keepdims=True)
    row = jax.lax.broadcasted_iota(jnp.int32, (V, V), 0)
    col = jax.lax.broadcasted_iota(jnp.int32, (V, V), 1)
    adjod_ref[...] = jnp.where(row == col, 0.0, adjT)
    qvT_ref[...] = qvT
    bqrT_ref[...] = jax.lax.dot_general(br_ref[...], qvT, (((0,), (0,)), ((), ())),
                                        precision=_PREC)
    bquT_ref[...] = jax.lax.dot_general(bu_ref[...], qvT, (((0,), (0,)), ((), ())),
                                        precision=_PREC)
    bqcT_ref[...] = jax.lax.dot_general(bc_ref[...], qvT, (((0,), (0,)), ((), ())),
                                        precision=_PREC)
    ivto_ref[...] = 1.0 / (jnp.sum(mask_ref[...], axis=1) + 1.0)


def _main_body(obs_ref, mask_ref, avg_ref, len_ref, rWT_ref,
               adjod_ref, qvT_ref, bqrT_ref, bquT_ref, bqcT_ref,
               Wru_ref, Wc_ref, ivto_ref, eye_ref, out_ref, h_ref):
    t = pl.program_id(0)

    @pl.when(t == 0)
    def _init():
        h_ref[...] = jnp.zeros((B, F, V), jnp.float32)
        out_ref[...] = jnp.zeros((B, F, V), jnp.float32)

    h = h_ref[...]
    obs_raw = obs_ref[...].reshape(B, V, F)
    mask_t = mask_ref[...].reshape(B, V)
    avg_t = avg_ref[...].reshape(B, V)
    rarity = ALPHA * jnp.tanh(avg_t * ivto_ref[...])
    dif = jnp.abs(rarity[:, :, None] - rarity[:, None, :])
    g = adjod_ref[...][None] * (1.0 - rWT_ref[...][None] * dif)
    mlane = mask_t[:, None, :]
    g = g * mask_t[:, :, None] * mlane
    obs_T = jax.lax.dot_general(
        obs_raw, eye_ref[...], (((1,), (1,)), ((0,), (0,))), precision=_PREC)
    c_obs = jax.lax.dot_general(
        obs_raw, g, (((1,), (1,)), ((0,), (0,))), precision=_PREC) + obs_T
    rh_part = jnp.concatenate([rarity[:, None, :], h], axis=1)
    c_rh = jax.lax.dot_general(
        rh_part, g, (((2,), (1,)), ((0,), (0,))), precision=_PREC) + rh_part
    obs_t = obs_T
    c = jnp.concatenate([c_obs, c_rh], axis=1)
    tmp_ru = jax.lax.dot_general(
        Wru_ref[...], c, (((2,), (1,)), ((0,), (0,))), precision=_PREC)
    qvt = qvT_ref[...]
    pre_r = bqrT_ref[...][None]
    pre_u = bquT_ref[...][None]
    for d in range(QD):
        q = qvt[d:d + 1]
        pre_r = pre_r + q * tmp_ru[:, d * F:(d + 1) * F, :]
        pre_u = pre_u + q * tmp_ru[:, QD * F + d * F:QD * F + (d + 1) * F, :]
    r = jax.nn.sigmoid(pre_r)
    u = jax.nn.sigmoid(pre_u)
    rh = r * h
    xh2 = jnp.concatenate([obs_t, rarity[:, None, :], rh], axis=1)
    tmp_c = jax.lax.dot_general(
        Wc_ref[...], xh2, (((2,), (1,)), ((0,), (0,))), precision=_PREC)
    pre_c = bqcT_ref[...][None]
    for d in range(QD):
        pre_c = pre_c + qvt[d:d + 1] * tmp_c[:, d * F:(d + 1) * F, :]
    cand = jnp.tanh(pre_c)
    m3 = mlane > 0.0
    h_new = jnp.where(m3, (1.0 - u) * rh + u * cand, h)
    h_ref[...] = h_new
    lenf3 = len_ref[...].astype(jnp.float32)[:, :, None]
    end = lenf3 == (t + 1).astype(jnp.float32)
    out_ref[...] = jnp.where(end, h_new, out_ref[...])


@jax.jit
def kernel(obs_emb, observed_mask, lengths, avg_interval, rarity_W,
           var_plm_rep, f_W1, f_b1, f_W2, f_b2, g_W1, g_b1, g_W2, g_b2,
           W_r, b_r, W_u, b_u, W_c, b_c):
    adjod, qvT, bqrT, bquT, bqcT, ivto = pl.pallas_call(
        _prelude_body,
        out_shape=[
            jax.ShapeDtypeStruct((V, V), jnp.float32),
            jax.ShapeDtypeStruct((QD, V), jnp.float32),
            jax.ShapeDtypeStruct((F, V), jnp.float32),
            jax.ShapeDtypeStruct((F, V), jnp.float32),
            jax.ShapeDtypeStruct((F, V), jnp.float32),
            jax.ShapeDtypeStruct((B, V), jnp.float32),
        ],
    )(var_plm_rep.T, f_W1.T, f_b1.reshape(H2, 1), f_W2.T, f_b2.reshape(QD, 1),
      g_W1.T, g_b1.reshape(H2, 1), g_W2.T, g_b2.reshape(NE, 1),
      b_r, b_u, b_c, observed_mask)

    WruT = jnp.concatenate([
        W_r.transpose(0, 2, 1).reshape(QD * F, IN),
        W_u.transpose(0, 2, 1).reshape(QD * F, IN)], axis=0)
    Wru_b = jnp.broadcast_to(WruT[None], (B, 2 * QD * F, IN))
    WcT = W_c.transpose(0, 2, 1).reshape(QD * F, IN)
    Wc_b = jnp.broadcast_to(WcT[None], (B, QD * F, IN))

    mask_T = observed_mask.transpose(1, 0, 2)
    avg_T = avg_interval.transpose(1, 0, 2)
    eye_b = jnp.broadcast_to(jnp.eye(V, dtype=jnp.float32)[None], (B, V, V))

    full = lambda shp: pl.BlockSpec(shp, lambda t: (0,) * len(shp))
    out_T = pl.pallas_call(
        _main_body,
        grid=(T,),
        in_specs=[
            pl.BlockSpec((B, 1, V, F), lambda t: (0, t, 0, 0)),
            pl.BlockSpec((1, B, V), lambda t: (t, 0, 0)),
            pl.BlockSpec((1, B, V), lambda t: (t, 0, 0)),
            full((B, 1)),
            full((V, V)),
            full((V, V)),
            full((QD, 1, V)),
            full((F, V)),
            full((F, V)),
            full((F, V)),
            full((B, 2 * QD * F, IN)),
            full((B, QD * F, IN)),
            full((B, V)),
            full((B, V, V)),
        ],
        out_specs=pl.BlockSpec((B, F, V), lambda t: (0, 0, 0)),
        out_shape=jax.ShapeDtypeStruct((B, F, V), jnp.float32),
        scratch_shapes=[pltpu.VMEM((B, F, V), jnp.float32)],
    )(obs_emb, mask_T, avg_T, lengths, rarity_W.T,
      adjod, qvT.reshape(QD, 1, V), bqrT, bquT, bqcT, Wru_b, Wc_b, ivto, eye_b)
    return out_T.transpose(0, 2, 1)


# speedup vs baseline: 1.1850x; 1.1850x over previous
import jax
import jax.numpy as jnp
from jax.experimental import pallas as pl
from jax.experimental.pallas import tpu as pltpu

B, T, V, F = 64, 48, 64, 16
QD, NE, PLM = 5, 16, 768
H2 = 2 * F
IN = 2 * F + 1
ALPHA = 1.0

_PREC = jax.lax.Precision.DEFAULT


def _prelude_body(vprT_ref, fW1T_ref, fb1_ref, fW2T_ref, fb2_ref,
                  gW1T_ref, gb1_ref, gW2T_ref, gb2_ref,
                  br_ref, bu_ref, bc_ref, mask_ref,
                  adjod_ref, qvT_ref, bqrT_ref, bquT_ref, bqcT_ref, ivto_ref):
    vprT = vprT_ref[...]
    hfT = jax.nn.relu(jnp.dot(fW1T_ref[...], vprT, precision=_PREC) + fb1_ref[...])
    qvT = jnp.dot(fW2T_ref[...], hfT, precision=_PREC) + fb2_ref[...]
    hgT = jax.nn.relu(jnp.dot(gW1T_ref[...], vprT, precision=_PREC) + gb1_ref[...])
    neT = jnp.dot(gW2T_ref[...], hgT, precision=_PREC) + gb2_ref[...]
    nrm = jnp.maximum(jnp.sqrt(jnp.sum(neT * neT, axis=0, keepdims=True)), 1e-12)
    neT = neT / nrm
    s = jax.lax.dot_general(neT, neT, (((0,), (0,)), ((), ())),
                            precision=_PREC)
    s = s - jnp.max(s, axis=0, keepdims=True)
    e = jnp.exp(s)
    adjT = e / jnp.sum(e, axis=0, keepdims=True)
    row = jax.lax.broadcasted_iota(jnp.int32, (V, V), 0)
    col = jax.lax.broadcasted_iota(jnp.int32, (V, V), 1)
    adjod_ref[...] = jnp.where(row == col, 0.0, adjT)
    qvT_ref[...] = qvT
    bqrT_ref[...] = jax.lax.dot_general(br_ref[...], qvT, (((0,), (0,)), ((), ())),
                                        precision=_PREC)
    bquT_ref[...] = jax.lax.dot_general(bu_ref[...], qvT, (((0,), (0,)), ((), ())),
                                        precision=_PREC)
    bqcT_ref[...] = jax.lax.dot_general(bc_ref[...], qvT, (((0,), (0,)), ((), ())),
                                        precision=_PREC)
    ivto_ref[...] = 1.0 / (jnp.sum(mask_ref[...], axis=1) + 1.0)


def _main_body(obs_ref, mask_ref, avg_ref, len_ref, rWT_ref,
               adjod_ref, qvT_ref, bqrT_ref, bquT_ref, bqcT_ref,
               Wru_ref, Wc_ref, ivto_ref, out_ref, h_ref):
    t = pl.program_id(0)

    @pl.when(t == 0)
    def _init():
        h_ref[...] = jnp.zeros((B, F, V), jnp.float32)
        out_ref[...] = jnp.zeros((B, F, V), jnp.float32)

    h = h_ref[...]
    obs_t = jnp.transpose(obs_ref[...].reshape(B, V, F), (0, 2, 1))
    mask_t = mask_ref[...].reshape(B, V)
    avg_t = avg_ref[...].reshape(B, V)
    rarity = ALPHA * jnp.tanh(avg_t * ivto_ref[...])
    dif = jnp.abs(rarity[:, :, None] - rarity[:, None, :])
    g = adjod_ref[...][None] * (1.0 - rWT_ref[...][None] * dif)
    mlane = mask_t[:, None, :]
    xh = jnp.concatenate([obs_t, rarity[:, None, :], h], axis=1)
    xhm = xh * mlane
    c = jax.lax.dot_general(
        xhm, g, (((2,), (1,)), ((0,), (0,))), precision=_PREC) * mlane + xh
    tmp_ru = jax.lax.dot_general(
        Wru_ref[...], c, (((2,), (1,)), ((0,), (0,))), precision=_PREC)
    qvt = qvT_ref[...]
    pre_r = bqrT_ref[...][None]
    pre_u = bquT_ref[...][None]
    for d in range(QD):
        q = qvt[d:d + 1]
        pre_r = pre_r + q * tmp_ru[:, d * F:(d + 1) * F, :]
        pre_u = pre_u + q * tmp_ru[:, QD * F + d * F:QD * F + (d + 1) * F, :]
    r = jax.nn.sigmoid(pre_r)
    u = jax.nn.sigmoid(pre_u)
    rh = r * h
    xh2 = jnp.concatenate([obs_t, rarity[:, None, :], rh], axis=1)
    tmp_c = jax.lax.dot_general(
        Wc_ref[...], xh2, (((2,), (1,)), ((0,), (0,))), precision=_PREC)
    pre_c = bqcT_ref[...][None]
    for d in range(QD):
        pre_c = pre_c + qvt[d:d + 1] * tmp_c[:, d * F:(d + 1) * F, :]
    cand = jnp.tanh(pre_c)
    m3 = mlane > 0.0
    h_new = jnp.where(m3, (1.0 - u) * rh + u * cand, h)
    h_ref[...] = h_new
    lenf3 = len_ref[...].astype(jnp.float32)[:, :, None]
    end = lenf3 == (t + 1).astype(jnp.float32)
    out_ref[...] = jnp.where(end, h_new, out_ref[...])


@jax.jit
def kernel(obs_emb, observed_mask, lengths, avg_interval, rarity_W,
           var_plm_rep, f_W1, f_b1, f_W2, f_b2, g_W1, g_b1, g_W2, g_b2,
           W_r, b_r, W_u, b_u, W_c, b_c):
    adjod, qvT, bqrT, bquT, bqcT, ivto = pl.pallas_call(
        _prelude_body,
        out_shape=[
            jax.ShapeDtypeStruct((V, V), jnp.float32),
            jax.ShapeDtypeStruct((QD, V), jnp.float32),
            jax.ShapeDtypeStruct((F, V), jnp.float32),
            jax.ShapeDtypeStruct((F, V), jnp.float32),
            jax.ShapeDtypeStruct((F, V), jnp.float32),
            jax.ShapeDtypeStruct((B, V), jnp.float32),
        ],
    )(var_plm_rep.T, f_W1.T, f_b1.reshape(H2, 1), f_W2.T, f_b2.reshape(QD, 1),
      g_W1.T, g_b1.reshape(H2, 1), g_W2.T, g_b2.reshape(NE, 1),
      b_r, b_u, b_c, observed_mask)

    WruT = jnp.concatenate([
        W_r.transpose(0, 2, 1).reshape(QD * F, IN),
        W_u.transpose(0, 2, 1).reshape(QD * F, IN)], axis=0)
    Wru_b = jnp.broadcast_to(WruT[None], (B, 2 * QD * F, IN))
    WcT = W_c.transpose(0, 2, 1).reshape(QD * F, IN)
    Wc_b = jnp.broadcast_to(WcT[None], (B, QD * F, IN))

    mask_T = observed_mask.transpose(1, 0, 2)
    avg_T = avg_interval.transpose(1, 0, 2)

    full = lambda shp: pl.BlockSpec(shp, lambda t: (0,) * len(shp))
    out_T = pl.pallas_call(
        _main_body,
        grid=(T,),
        in_specs=[
            pl.BlockSpec((B, 1, V, F), lambda t: (0, t, 0, 0)),
            pl.BlockSpec((1, B, V), lambda t: (t, 0, 0)),
            pl.BlockSpec((1, B, V), lambda t: (t, 0, 0)),
            full((B, 1)),
            full((V, V)),
            full((V, V)),
            full((QD, 1, V)),
            full((F, V)),
            full((F, V)),
            full((F, V)),
            full((B, 2 * QD * F, IN)),
            full((B, QD * F, IN)),
            full((B, V)),
        ],
        out_specs=pl.BlockSpec((B, F, V), lambda t: (0, 0, 0)),
        out_shape=jax.ShapeDtypeStruct((B, F, V), jnp.float32),
        scratch_shapes=[pltpu.VMEM((B, F, V), jnp.float32)],
    )(obs_emb, mask_T, avg_T, lengths, rarity_W.T,
      adjod, qvT.reshape(QD, 1, V), bqrT, bquT, bqcT, Wru_b, Wc_b, ivto)
    return out_T.transpose(0, 2, 1)


# speedup vs baseline: 1.8066x; 1.5246x over previous
import jax
import jax.numpy as jnp
from jax.experimental import pallas as pl
from jax.experimental.pallas import tpu as pltpu

B, T, V, F = 64, 48, 64, 16
QD, NE, PLM = 5, 16, 768
H2 = 2 * F
IN = 2 * F + 1
ALPHA = 1.0

_PREC = jax.lax.Precision.DEFAULT


def _prelude_body(vprT_ref, fW1T_ref, fb1_ref, fW2T_ref, fb2_ref,
                  gW1T_ref, gb1_ref, gW2T_ref, gb2_ref,
                  br_ref, bu_ref, bc_ref, mask_ref,
                  adjod_ref, qvT_ref, bqrT_ref, bquT_ref, bqcT_ref, ivto_ref):
    vprT = vprT_ref[...]
    hfT = jax.nn.relu(jnp.dot(fW1T_ref[...], vprT, precision=_PREC) + fb1_ref[...])
    qvT = jnp.dot(fW2T_ref[...], hfT, precision=_PREC) + fb2_ref[...]
    hgT = jax.nn.relu(jnp.dot(gW1T_ref[...], vprT, precision=_PREC) + gb1_ref[...])
    neT = jnp.dot(gW2T_ref[...], hgT, precision=_PREC) + gb2_ref[...]
    nrm = jnp.maximum(jnp.sqrt(jnp.sum(neT * neT, axis=0, keepdims=True)), 1e-12)
    neT = neT / nrm
    s = jax.lax.dot_general(neT, neT, (((0,), (0,)), ((), ())),
                            precision=_PREC)
    s = s - jnp.max(s, axis=0, keepdims=True)
    e = jnp.exp(s)
    adjT = e / jnp.sum(e, axis=0, keepdims=True)
    row = jax.lax.broadcasted_iota(jnp.int32, (V, V), 0)
    col = jax.lax.broadcasted_iota(jnp.int32, (V, V), 1)
    adjod_ref[...] = jnp.where(row == col, 0.0, adjT)
    qvT_ref[...] = qvT
    bqrT_ref[...] = jax.lax.dot_general(br_ref[...], qvT, (((0,), (0,)), ((), ())),
                                        precision=_PREC)
    bquT_ref[...] = jax.lax.dot_general(bu_ref[...], qvT, (((0,), (0,)), ((), ())),
                                        precision=_PREC)
    bqcT_ref[...] = jax.lax.dot_general(bc_ref[...], qvT, (((0,), (0,)), ((), ())),
                                        precision=_PREC)
    ivto_ref[...] = 1.0 / (jnp.sum(mask_ref[...], axis=1) + 1.0)


def _main_body(obs_ref, mask_ref, avg_ref, len_ref, rWT_ref,
               adjod_ref, qvT_ref, bqrT_ref, bquT_ref, bqcT_ref,
               Wru_ref, Wc_ref, ivto_ref, out_ref, h_ref):
    t = pl.program_id(0)

    @pl.when(t == 0)
    def _init():
        h_ref[...] = jnp.zeros((B, F, V), jnp.float32)
        out_ref[...] = jnp.zeros((B, F, V), jnp.float32)

    h = h_ref[...]
    obs_t = obs_ref[...].reshape(B, F, V)
    mask_t = mask_ref[...].reshape(B, V)
    avg_t = avg_ref[...].reshape(B, V)
    rarity = ALPHA * jnp.tanh(avg_t * ivto_ref[...])
    dif = jnp.abs(rarity[:, :, None] - rarity[:, None, :])
    g = adjod_ref[...][None] * (1.0 - rWT_ref[...][None] * dif)
    mlane = mask_t[:, None, :]
    xh = jnp.concatenate([obs_t, h, rarity[:, None, :]], axis=1)
    xhm = xh * mlane
    c = jax.lax.dot_general(
        xhm, g, (((2,), (1,)), ((0,), (0,))), precision=_PREC) * mlane + xh
    tmp_ru = jax.lax.dot_general(
        Wru_ref[...], c, (((2,), (1,)), ((0,), (0,))), precision=_PREC)
    qvt = qvT_ref[...]
    pre_r = bqrT_ref[...][None]
    pre_u = bquT_ref[...][None]
    for d in range(QD):
        q = qvt[d:d + 1]
        pre_r = pre_r + q * tmp_ru[:, d * F:(d + 1) * F, :]
        pre_u = pre_u + q * tmp_ru[:, QD * F + d * F:QD * F + (d + 1) * F, :]
    r = jax.nn.sigmoid(pre_r)
    u = jax.nn.sigmoid(pre_u)
    rh = r * h
    xh2 = jnp.concatenate([obs_t, rh, rarity[:, None, :]], axis=1)
    tmp_c = jax.lax.dot_general(
        Wc_ref[...], xh2, (((2,), (1,)), ((0,), (0,))), precision=_PREC)
    pre_c = bqcT_ref[...][None]
    for d in range(QD):
        pre_c = pre_c + qvt[d:d + 1] * tmp_c[:, d * F:(d + 1) * F, :]
    cand = jnp.tanh(pre_c)
    m3 = mlane > 0.0
    h_new = jnp.where(m3, (1.0 - u) * rh + u * cand, h)
    h_ref[...] = h_new
    lenf3 = len_ref[...].astype(jnp.float32)[:, :, None]
    end = lenf3 == (t + 1).astype(jnp.float32)
    out_ref[...] = jnp.where(end, h_new, out_ref[...])


@jax.jit
def kernel(obs_emb, observed_mask, lengths, avg_interval, rarity_W,
           var_plm_rep, f_W1, f_b1, f_W2, f_b2, g_W1, g_b1, g_W2, g_b2,
           W_r, b_r, W_u, b_u, W_c, b_c):
    adjod, qvT, bqrT, bquT, bqcT, ivto = pl.pallas_call(
        _prelude_body,
        out_shape=[
            jax.ShapeDtypeStruct((V, V), jnp.float32),
            jax.ShapeDtypeStruct((QD, V), jnp.float32),
            jax.ShapeDtypeStruct((F, V), jnp.float32),
            jax.ShapeDtypeStruct((F, V), jnp.float32),
            jax.ShapeDtypeStruct((F, V), jnp.float32),
            jax.ShapeDtypeStruct((B, V), jnp.float32),
        ],
    )(var_plm_rep.T, f_W1.T, f_b1.reshape(H2, 1), f_W2.T, f_b2.reshape(QD, 1),
      g_W1.T, g_b1.reshape(H2, 1), g_W2.T, g_b2.reshape(NE, 1),
      b_r, b_u, b_c, observed_mask)

    perm = jnp.array(list(range(16)) + list(range(17, 33)) + [16])
    WruT = jnp.concatenate([
        W_r.transpose(0, 2, 1).reshape(QD * F, IN),
        W_u.transpose(0, 2, 1).reshape(QD * F, IN)], axis=0)[:, perm]
    Wru_b = jnp.broadcast_to(WruT[None], (B, 2 * QD * F, IN))
    WcT = W_c.transpose(0, 2, 1).reshape(QD * F, IN)[:, perm]
    Wc_b = jnp.broadcast_to(WcT[None], (B, QD * F, IN))

    obs_T = obs_emb.transpose(1, 0, 3, 2)
    mask_T = observed_mask.transpose(1, 0, 2)
    avg_T = avg_interval.transpose(1, 0, 2)

    full = lambda shp: pl.BlockSpec(shp, lambda t: (0,) * len(shp))
    out_T = pl.pallas_call(
        _main_body,
        grid=(T,),
        in_specs=[
            pl.BlockSpec((1, B, F, V), lambda t: (t, 0, 0, 0)),
            pl.BlockSpec((1, B, V), lambda t: (t, 0, 0)),
            pl.BlockSpec((1, B, V), lambda t: (t, 0, 0)),
            full((B, 1)),
            full((V, V)),
            full((V, V)),
            full((QD, 1, V)),
            full((F, V)),
            full((F, V)),
            full((F, V)),
            full((B, 2 * QD * F, IN)),
            full((B, QD * F, IN)),
            full((B, V)),
        ],
        out_specs=pl.BlockSpec((B, F, V), lambda t: (0, 0, 0)),
        out_shape=jax.ShapeDtypeStruct((B, F, V), jnp.float32),
        scratch_shapes=[pltpu.VMEM((B, F, V), jnp.float32)],
    )(obs_T, mask_T, avg_T, lengths, rarity_W.T,
      adjod, qvT.reshape(QD, 1, V), bqrT, bquT, bqcT, Wru_b, Wc_b, ivto)
    return out_T.transpose(0, 2, 1)
